# Initial kernel scaffold; baseline (speedup 1.0000x reference)
import jax
import jax.numpy as jnp
from jax.experimental import pallas as pl


def kernel(x, pos_table):
    raise NotImplementedError("write your pallas kernel here")


# speedup vs baseline: 10.2550x; 10.2550x over previous
import functools

import jax
import jax.numpy as jnp
from jax import lax
from jax.experimental import pallas as pl
from jax.experimental.pallas import tpu as pltpu
from jax.experimental.pallas import tpu_sc as plsc


def _make_sc_broadcast(B, L, D, NC, NS):
    NW = NC * NS
    rows_per_w = B // NW
    row_words = L * D
    rep = 1
    for cand in range(min(rows_per_w, (120 * 1024) // row_words), 0, -1):
        if rows_per_w % cand == 0 and cand * row_words * 4 <= 480 * 1024:
            rep = cand
            break
    n_dma = rows_per_w // rep

    mesh = plsc.VectorSubcoreMesh(core_axis_name="c", subcore_axis_name="s")

    @functools.partial(
        pl.kernel,
        mesh=mesh,
        out_type=jax.ShapeDtypeStruct((B, row_words), jnp.float32),
        scratch_types=[
            pltpu.VMEM((rep, row_words), jnp.float32),
            pltpu.SemaphoreType.DMA,
        ],
    )
    def k(table_hbm, out_hbm, buf, sem):
        wid = lax.axis_index("s") * NC + lax.axis_index("c")
        base = wid * rows_per_w
        for r in range(rep):
            pltpu.sync_copy(table_hbm, buf.at[r])
        copies = [
            pltpu.async_copy(buf, out_hbm.at[pl.ds(base + i * rep, rep)], sem)
            for i in range(n_dma)
        ]
        for c in copies:
            c.wait()

    return k


def kernel(x, pos_table):
    B, L = x.shape
    D = pos_table.shape[1]
    info = plsc.get_sparse_core_info()
    NC, NS = info.num_cores, info.num_subcores
    table_flat = pos_table[:L].reshape(L * D)
    k = _make_sc_broadcast(B, L, D, NC, NS)
    out = k(table_flat)
    return out.reshape(B, L, D)
